# baseline (speedup 1.0000x reference)
import functools

import jax
import jax.numpy as jnp
from jax.experimental import pallas as pl
from jax.experimental.pallas import tpu as pltpu


def _support_body(x_ref, w_ref, out_ref):
    x = x_ref[...].astype(jnp.bfloat16)
    w = w_ref[...].astype(jnp.bfloat16)
    out_ref[...] = jnp.dot(
        x, w, preferred_element_type=jnp.float32
    ).astype(jnp.bfloat16)


def _spmm_body(adj_ref, sup_ref, b_ref, out_ref):
    a = adj_ref[...].astype(jnp.bfloat16)
    part = jnp.dot(a, sup_ref[...], preferred_element_type=jnp.float32)
    out_ref[...] = part + b_ref[...]


@functools.partial(jax.jit, static_argnames=())
def kernel(input, adj, W, b):
    n, in_f = input.shape
    out_f = W.shape[1]

    bm_sup = 2000 if n % 2000 == 0 else n
    support = pl.pallas_call(
        _support_body,
        grid=(n // bm_sup,),
        in_specs=[
            pl.BlockSpec((bm_sup, in_f), lambda i: (i, 0)),
            pl.BlockSpec((in_f, out_f), lambda i: (0, 0)),
        ],
        out_specs=pl.BlockSpec((bm_sup, out_f), lambda i: (i, 0)),
        out_shape=jax.ShapeDtypeStruct((n, out_f), jnp.bfloat16),
        compiler_params=pltpu.CompilerParams(
            dimension_semantics=("parallel",),
        ),
    )(input, W)

    bm = 576
    b2 = b.reshape(1, out_f)
    out = pl.pallas_call(
        _spmm_body,
        grid=(pl.cdiv(n, bm),),
        in_specs=[
            pl.BlockSpec((bm, n), lambda m: (m, 0)),
            pl.BlockSpec((n, out_f), lambda m: (0, 0)),
            pl.BlockSpec((1, out_f), lambda m: (0, 0)),
        ],
        out_specs=pl.BlockSpec((bm, out_f), lambda m: (m, 0)),
        out_shape=jax.ShapeDtypeStruct((n, out_f), jnp.float32),
        compiler_params=pltpu.CompilerParams(
            dimension_semantics=("parallel",),
        ),
    )(adj, support, b2)
    return out


# speedup vs baseline: 1.0353x; 1.0353x over previous
import functools

import jax
import jax.numpy as jnp
from jax.experimental import pallas as pl
from jax.experimental.pallas import tpu as pltpu

_NW = 10


def _fused_body(x_ref, w_ref, adj_ref, b_ref, out_ref, sup_ref):
    m = pl.program_id(0)

    @pl.when(m < _NW)
    def _support_chunk():
        chunk = x_ref.shape[0]
        x = x_ref[...].astype(jnp.bfloat16)
        w = w_ref[...].astype(jnp.bfloat16)
        sup_ref[pl.ds(m * chunk, chunk), :] = jnp.dot(
            x, w, preferred_element_type=jnp.float32
        ).astype(jnp.bfloat16)

    @pl.when(m >= _NW)
    def _spmm():
        a = adj_ref[...].astype(jnp.bfloat16)
        part = jnp.dot(a, sup_ref[...], preferred_element_type=jnp.float32)
        out_ref[...] = part + b_ref[...]


@functools.partial(jax.jit, static_argnames=())
def kernel(input, adj, W, b):
    n, in_f = input.shape
    out_f = W.shape[1]

    chunk = n // _NW
    bm = 512
    n_spmm = pl.cdiv(n, bm)
    b2 = b.reshape(1, out_f)

    out = pl.pallas_call(
        _fused_body,
        grid=(_NW + n_spmm,),
        in_specs=[
            pl.BlockSpec((chunk, in_f), lambda m: (jnp.minimum(m, _NW - 1), 0)),
            pl.BlockSpec((in_f, out_f), lambda m: (0, 0)),
            pl.BlockSpec((bm, n), lambda m: (jnp.maximum(m - _NW, 0), 0)),
            pl.BlockSpec((1, out_f), lambda m: (0, 0)),
        ],
        out_specs=pl.BlockSpec((bm, out_f), lambda m: (jnp.maximum(m - _NW, 0), 0)),
        out_shape=jax.ShapeDtypeStruct((n, out_f), jnp.float32),
        scratch_shapes=[pltpu.VMEM((n, out_f), jnp.bfloat16)],
        compiler_params=pltpu.CompilerParams(
            dimension_semantics=("arbitrary",),
        ),
    )(input, W, adj, b2)
    return out
